# baseline (speedup 1.0000x reference)
import functools

import jax
import jax.numpy as jnp
from jax import lax
from jax.experimental import pallas as pl
from jax.experimental.pallas import tpu as pltpu
from jax.experimental.pallas import tpu_sc as plsc

N_NODES = 10000
N_EDGES = 320000
NC, NS = 2, 16
NW = NC * NS
EPW = N_EDGES // NW
CHUNK = 100
NITER = EPW // CHUNK
QUAD = 4
NQD = NITER // QUAD
PAIR = 3
NPAD = 10240
RPT = NPAD // NS
BR = 400


def _sc_degrees(srcr, dstr, ones_c, zeros_r):
    mesh = plsc.VectorSubcoreMesh(core_axis_name="c", subcore_axis_name="s")

    @functools.partial(
        pl.kernel,
        out_type=jax.ShapeDtypeStruct((NC, 2, NPAD), jnp.float32),
        mesh=mesh,
        scratch_types=[
            pltpu.VMEM((NITER, CHUNK), jnp.int32),
            pltpu.VMEM((NITER, CHUNK), jnp.int32),
            pltpu.VMEM((CHUNK,), jnp.float32),
            pltpu.VMEM_SHARED((NPAD,), jnp.float32),
            pltpu.VMEM_SHARED((NPAD,), jnp.float32),
        ] + [pltpu.SemaphoreType.DMA] * (2 * QUAD),
    )
    def deg_kernel(srcr_hbm, dstr_hbm, ones_hbm, zeros_hbm, out_hbm,
                   sidx, didx, ones_v, cs, cd, *sems):
        cid = lax.axis_index("c")
        sid = lax.axis_index("s")
        wid = sid * NC + cid
        pltpu.sync_copy(ones_hbm, ones_v)
        pltpu.sync_copy(srcr_hbm.at[wid], sidx)
        pltpu.sync_copy(dstr_hbm.at[wid], didx)
        pltpu.sync_copy(zeros_hbm, cs.at[pl.ds(sid * RPT, RPT)])
        pltpu.sync_copy(zeros_hbm, cd.at[pl.ds(sid * RPT, RPT)])
        plsc.subcore_barrier()

        def body(q, _):
            descs = []
            for k in range(QUAD):
                i = q * QUAD + k
                descs.append(pltpu.async_copy(
                    ones_v, cs.at[sidx.at[i]], sems[k], add=True))
                descs.append(pltpu.async_copy(
                    ones_v, cd.at[didx.at[i]], sems[QUAD + k], add=True))
            for de in descs:
                de.wait()
            return ()

        lax.fori_loop(0, NQD, body, ())
        plsc.subcore_barrier()
        pltpu.sync_copy(cs.at[pl.ds(sid * RPT, RPT)],
                        out_hbm.at[cid, 0, pl.ds(sid * RPT, RPT)])
        pltpu.sync_copy(cd.at[pl.ds(sid * RPT, RPT)],
                        out_hbm.at[cid, 1, pl.ds(sid * RPT, RPT)])

    return deg_kernel(srcr, dstr, ones_c, zeros_r)


def _sc_aggregate(h, srcr, dstr, zeros_rows, d):
    mesh = plsc.VectorSubcoreMesh(core_axis_name="c", subcore_axis_name="s")

    @functools.partial(
        pl.kernel,
        out_type=jax.ShapeDtypeStruct((NC, NPAD, d), jnp.float32),
        mesh=mesh,
        scratch_types=[
            pltpu.VMEM((PAIR, CHUNK), jnp.int32),
            pltpu.VMEM((PAIR, CHUNK), jnp.int32),
            pltpu.VMEM((PAIR, CHUNK, d), jnp.float32),
            pltpu.VMEM_SHARED((NPAD, d), jnp.float32),
            pltpu.SemaphoreType.DMA((PAIR,)),
            pltpu.SemaphoreType.DMA((PAIR,)),
            pltpu.SemaphoreType.DMA,
            pltpu.SemaphoreType.DMA,
        ],
    )
    def agg_kernel(h_hbm, srcr_hbm, dstr_hbm, zeros_hbm, out_hbm,
                   sidx, didx, rows, acc, gsem, ssem, isem_s, isem_d):
        cid = lax.axis_index("c")
        sid = lax.axis_index("s")
        wid = sid * NC + cid
        pltpu.sync_copy(zeros_hbm, acc.at[pl.ds(sid * RPT, RPT)])
        plsc.subcore_barrier()

        def gather(i, b):
            return pltpu.async_copy(
                h_hbm.at[sidx.at[b]], rows.at[b], gsem.at[b])

        def scatter(b):
            return pltpu.async_copy(
                rows.at[b], acc.at[didx.at[b]], ssem.at[b], add=True)

        def load_idx(i, b):
            pltpu.async_copy(srcr_hbm.at[wid, i], sidx.at[b], isem_s)
            pltpu.async_copy(dstr_hbm.at[wid, i], didx.at[b], isem_d)

        def wait_idx(b):
            pltpu.make_async_copy(srcr_hbm.at[wid, 0], sidx.at[b],
                                  isem_s).wait()
            pltpu.make_async_copy(dstr_hbm.at[wid, 0], didx.at[b],
                                  isem_d).wait()

        pltpu.sync_copy(srcr_hbm.at[wid, 0], sidx.at[0])
        pltpu.sync_copy(dstr_hbm.at[wid, 0], didx.at[0])
        gather(0, 0)

        def body(i, _):
            b = i % PAIR
            nb = (i + 1) % PAIR

            @pl.when(i >= PAIR - 1)
            def _():
                pltpu.make_async_copy(rows.at[nb], acc.at[didx.at[nb]],
                                      ssem.at[nb]).wait()

            @pl.when(i + 1 < NITER)
            def _():
                load_idx(i + 1, nb)

            pltpu.make_async_copy(h_hbm.at[sidx.at[b]], rows.at[b],
                                  gsem.at[b]).wait()
            scatter(b)

            @pl.when(i + 1 < NITER)
            def _():
                wait_idx(nb)
                gather(i + 1, nb)

            return ()

        lax.fori_loop(0, NITER, body, ())
        for j in range(NITER - PAIR + 1, NITER):
            pltpu.make_async_copy(rows.at[j % PAIR],
                                  acc.at[didx.at[j % PAIR]],
                                  ssem.at[j % PAIR]).wait()
        plsc.subcore_barrier()
        pltpu.sync_copy(acc.at[pl.ds(sid * RPT, RPT)],
                        out_hbm.at[cid, pl.ds(sid * RPT, RPT)])

    return agg_kernel(h, srcr, dstr, zeros_rows)


def _norm_src(c):
    return lax.rsqrt(jnp.maximum(c[:, 0:1] + c[:, 2:3], 1.0))


def _norm_dst(c):
    return lax.rsqrt(jnp.maximum(c[:, 1:2] + c[:, 3:4], 1.0))


def _tc_linear1(x, cnt, w):
    d_in, d_out = w.shape

    def body(x_ref, c_ref, w_ref, o_ref):
        ns = _norm_src(c_ref[...])
        o_ref[...] = jnp.dot(x_ref[...] * ns, w_ref[...],
                             preferred_element_type=jnp.float32)

    return pl.pallas_call(
        body,
        grid=(N_NODES // BR,),
        in_specs=[
            pl.BlockSpec((BR, d_in), lambda i: (i, 0)),
            pl.BlockSpec((BR, 4), lambda i: (i, 0)),
            pl.BlockSpec((d_in, d_out), lambda i: (0, 0)),
        ],
        out_specs=pl.BlockSpec((BR, d_out), lambda i: (i, 0)),
        out_shape=jax.ShapeDtypeStruct((N_NODES, d_out), jnp.float32),
    )(x, cnt, w)


def _tc_mid(p, cnt, b1, w2):
    d_in, d_out = w2.shape

    def body(p_ref, c_ref, b_ref, w_ref, o_ref):
        c = c_ref[...]
        x = (p_ref[0] + p_ref[1]) * _norm_dst(c) + b_ref[...]
        x = jnp.maximum(x, 0.0)
        o_ref[...] = jnp.dot(x * _norm_src(c), w_ref[...],
                             preferred_element_type=jnp.float32)

    return pl.pallas_call(
        body,
        grid=(N_NODES // BR,),
        in_specs=[
            pl.BlockSpec((NC, BR, d_in), lambda i: (0, i, 0)),
            pl.BlockSpec((BR, 4), lambda i: (i, 0)),
            pl.BlockSpec((1, d_in), lambda i: (0, 0)),
            pl.BlockSpec((d_in, d_out), lambda i: (0, 0)),
        ],
        out_specs=pl.BlockSpec((BR, d_out), lambda i: (i, 0)),
        out_shape=jax.ShapeDtypeStruct((N_NODES, d_out), jnp.float32),
    )(p, cnt, b1, w2)


def _tc_out(q, cnt, b2, d_out):
    d = q.shape[-1]

    def body(q_ref, c_ref, b_ref, o_ref):
        t = (q_ref[0] + q_ref[1]) * _norm_dst(c_ref[...])
        o_ref[...] = t[:, :d_out] + b_ref[...]

    return pl.pallas_call(
        body,
        grid=(N_NODES // BR,),
        in_specs=[
            pl.BlockSpec((NC, BR, d), lambda i: (0, i, 0)),
            pl.BlockSpec((BR, 4), lambda i: (i, 0)),
            pl.BlockSpec((1, d_out), lambda i: (0, 0)),
        ],
        out_specs=pl.BlockSpec((BR, d_out), lambda i: (i, 0)),
        out_shape=jax.ShapeDtypeStruct((N_NODES, d_out), jnp.float32),
    )(q, cnt, b2)


def kernel(g, features, W1, b1, W2, b2):
    src = g[0].reshape(NW, NITER, CHUNK)
    dst = g[1].reshape(NW, NITER, CHUNK)
    nhid = W1.shape[1]
    nlabel = W2.shape[1]

    ones_c = jnp.ones((CHUNK,), jnp.float32)
    zeros_r = jnp.zeros((RPT,), jnp.float32)
    zeros_h = jnp.zeros((RPT, nhid), jnp.float32)

    w2p = jnp.pad(W2, ((0, 0), (0, nhid - nlabel)))

    counts = _sc_degrees(src, dst, ones_c, zeros_r)
    cnt = counts.reshape(4, NPAD).T

    h1 = _tc_linear1(features, cnt, W1)
    p = _sc_aggregate(h1, src, dst, zeros_h, nhid)
    h2 = _tc_mid(p, cnt, b1.reshape(1, nhid), w2p)
    q = _sc_aggregate(h2, src, dst, zeros_h, nhid)
    return _tc_out(q, cnt, b2.reshape(1, nlabel), nlabel)


# speedup vs baseline: 1.1262x; 1.1262x over previous
import functools

import jax
import jax.numpy as jnp
from jax import lax
from jax.experimental import pallas as pl
from jax.experimental.pallas import tpu as pltpu
from jax.experimental.pallas import tpu_sc as plsc

N_NODES = 10000
N_EDGES = 320000
NC, NS = 2, 16
NW = NC * NS
EPW = N_EDGES // NW
CHUNK = 125
NITER = EPW // CHUNK
QUAD = 4
NQD = NITER // QUAD
PAIR = 2
NPAD = 10240
RPT = NPAD // NS
BR = 400


def _sc_degrees(srcr, dstr, ones_c, zeros_r):
    mesh = plsc.VectorSubcoreMesh(core_axis_name="c", subcore_axis_name="s")

    @functools.partial(
        pl.kernel,
        out_type=jax.ShapeDtypeStruct((NC, 2, NPAD), jnp.float32),
        mesh=mesh,
        scratch_types=[
            pltpu.VMEM((NITER, CHUNK), jnp.int32),
            pltpu.VMEM((NITER, CHUNK), jnp.int32),
            pltpu.VMEM((CHUNK,), jnp.float32),
            pltpu.VMEM_SHARED((NPAD,), jnp.float32),
            pltpu.VMEM_SHARED((NPAD,), jnp.float32),
        ] + [pltpu.SemaphoreType.DMA] * (2 * QUAD),
    )
    def deg_kernel(srcr_hbm, dstr_hbm, ones_hbm, zeros_hbm, out_hbm,
                   sidx, didx, ones_v, cs, cd, *sems):
        cid = lax.axis_index("c")
        sid = lax.axis_index("s")
        wid = sid * NC + cid
        pltpu.sync_copy(ones_hbm, ones_v)
        pltpu.sync_copy(srcr_hbm.at[wid], sidx)
        pltpu.sync_copy(dstr_hbm.at[wid], didx)
        pltpu.sync_copy(zeros_hbm, cs.at[pl.ds(sid * RPT, RPT)])
        pltpu.sync_copy(zeros_hbm, cd.at[pl.ds(sid * RPT, RPT)])
        plsc.subcore_barrier()

        def body(q, _):
            descs = []
            for k in range(QUAD):
                i = q * QUAD + k
                descs.append(pltpu.async_copy(
                    ones_v, cs.at[sidx.at[i]], sems[k], add=True))
                descs.append(pltpu.async_copy(
                    ones_v, cd.at[didx.at[i]], sems[QUAD + k], add=True))
            for de in descs:
                de.wait()
            return ()

        lax.fori_loop(0, NQD, body, ())
        plsc.subcore_barrier()
        pltpu.sync_copy(cs.at[pl.ds(sid * RPT, RPT)],
                        out_hbm.at[cid, 0, pl.ds(sid * RPT, RPT)])
        pltpu.sync_copy(cd.at[pl.ds(sid * RPT, RPT)],
                        out_hbm.at[cid, 1, pl.ds(sid * RPT, RPT)])

    return deg_kernel(srcr, dstr, ones_c, zeros_r)


def _sc_aggregate(h, srcr, dstr, zeros_rows, d, linear_tiling=False):
    mesh = plsc.VectorSubcoreMesh(core_axis_name="c", subcore_axis_name="s")
    params = (pltpu.CompilerParams(use_tc_tiling_on_sc=False)
              if linear_tiling else None)

    @functools.partial(
        pl.kernel,
        out_type=jax.ShapeDtypeStruct((NC, NPAD, d), jnp.float32),
        mesh=mesh,
        compiler_params=params,
        scratch_types=[
            pltpu.VMEM((PAIR, CHUNK), jnp.int32),
            pltpu.VMEM((PAIR, CHUNK), jnp.int32),
            pltpu.VMEM((PAIR, CHUNK, d), jnp.float32),
            pltpu.VMEM_SHARED((NPAD, d), jnp.float32),
            pltpu.SemaphoreType.DMA((PAIR,)),
            pltpu.SemaphoreType.DMA((PAIR,)),
            pltpu.SemaphoreType.DMA,
            pltpu.SemaphoreType.DMA,
        ],
    )
    def agg_kernel(h_hbm, srcr_hbm, dstr_hbm, zeros_hbm, out_hbm,
                   sidx, didx, rows, acc, gsem, ssem, isem_s, isem_d):
        cid = lax.axis_index("c")
        sid = lax.axis_index("s")
        wid = sid * NC + cid
        pltpu.sync_copy(zeros_hbm, acc.at[pl.ds(sid * RPT, RPT)])
        plsc.subcore_barrier()

        def gather(i, b):
            return pltpu.async_copy(
                h_hbm.at[sidx.at[b]], rows.at[b], gsem.at[b])

        def scatter(b):
            return pltpu.async_copy(
                rows.at[b], acc.at[didx.at[b]], ssem.at[b], add=True)

        def load_idx(i, b):
            pltpu.async_copy(srcr_hbm.at[wid, i], sidx.at[b], isem_s)
            pltpu.async_copy(dstr_hbm.at[wid, i], didx.at[b], isem_d)

        def wait_idx(b):
            pltpu.make_async_copy(srcr_hbm.at[wid, 0], sidx.at[b],
                                  isem_s).wait()
            pltpu.make_async_copy(dstr_hbm.at[wid, 0], didx.at[b],
                                  isem_d).wait()

        pltpu.sync_copy(srcr_hbm.at[wid, 0], sidx.at[0])
        pltpu.sync_copy(dstr_hbm.at[wid, 0], didx.at[0])
        gather(0, 0)

        def body(i, _):
            b = i % PAIR
            nb = (i + 1) % PAIR

            @pl.when(i >= PAIR - 1)
            def _():
                pltpu.make_async_copy(rows.at[nb], acc.at[didx.at[nb]],
                                      ssem.at[nb]).wait()

            @pl.when(i + 1 < NITER)
            def _():
                load_idx(i + 1, nb)

            pltpu.make_async_copy(h_hbm.at[sidx.at[b]], rows.at[b],
                                  gsem.at[b]).wait()
            scatter(b)

            @pl.when(i + 1 < NITER)
            def _():
                wait_idx(nb)
                gather(i + 1, nb)

            return ()

        lax.fori_loop(0, NITER, body, ())
        for j in range(NITER - PAIR + 1, NITER):
            pltpu.make_async_copy(rows.at[j % PAIR],
                                  acc.at[didx.at[j % PAIR]],
                                  ssem.at[j % PAIR]).wait()
        plsc.subcore_barrier()
        pltpu.sync_copy(acc.at[pl.ds(sid * RPT, RPT)],
                        out_hbm.at[cid, pl.ds(sid * RPT, RPT)])

    return agg_kernel(h, srcr, dstr, zeros_rows)


def _norm_src(c):
    return lax.rsqrt(jnp.maximum(c[:, 0:1] + c[:, 2:3], 1.0))


def _norm_dst(c):
    return lax.rsqrt(jnp.maximum(c[:, 1:2] + c[:, 3:4], 1.0))


def _tc_linear1(x, cnt, w):
    d_in, d_out = w.shape

    def body(x_ref, c_ref, w_ref, o_ref):
        ns = _norm_src(c_ref[...])
        o_ref[...] = jnp.dot(x_ref[...] * ns, w_ref[...],
                             preferred_element_type=jnp.float32)

    return pl.pallas_call(
        body,
        grid=(N_NODES // BR,),
        in_specs=[
            pl.BlockSpec((BR, d_in), lambda i: (i, 0)),
            pl.BlockSpec((BR, 4), lambda i: (i, 0)),
            pl.BlockSpec((d_in, d_out), lambda i: (0, 0)),
        ],
        out_specs=pl.BlockSpec((BR, d_out), lambda i: (i, 0)),
        out_shape=jax.ShapeDtypeStruct((N_NODES, d_out), jnp.float32),
    )(x, cnt, w)


def _tc_mid(p, cnt, b1, w2):
    d_in, d_out = w2.shape

    def body(p_ref, c_ref, b_ref, w_ref, o_ref):
        c = c_ref[...]
        x = (p_ref[0] + p_ref[1]) * _norm_dst(c) + b_ref[...]
        x = jnp.maximum(x, 0.0)
        o_ref[...] = jnp.dot(x * _norm_src(c), w_ref[...],
                             preferred_element_type=jnp.float32)

    return pl.pallas_call(
        body,
        grid=(N_NODES // BR,),
        in_specs=[
            pl.BlockSpec((NC, BR, d_in), lambda i: (0, i, 0)),
            pl.BlockSpec((BR, 4), lambda i: (i, 0)),
            pl.BlockSpec((1, d_in), lambda i: (0, 0)),
            pl.BlockSpec((d_in, d_out), lambda i: (0, 0)),
        ],
        out_specs=pl.BlockSpec((BR, d_out), lambda i: (i, 0)),
        out_shape=jax.ShapeDtypeStruct((N_NODES, d_out), jnp.float32),
    )(p, cnt, b1, w2)


def _tc_out(q, cnt, b2, d_out):
    d = q.shape[-1]

    def body(q_ref, c_ref, b_ref, o_ref):
        t = (q_ref[0] + q_ref[1]) * _norm_dst(c_ref[...])
        o_ref[...] = t[:, :d_out] + b_ref[...]

    return pl.pallas_call(
        body,
        grid=(N_NODES // BR,),
        in_specs=[
            pl.BlockSpec((NC, BR, d), lambda i: (0, i, 0)),
            pl.BlockSpec((BR, 4), lambda i: (i, 0)),
            pl.BlockSpec((1, d_out), lambda i: (0, 0)),
        ],
        out_specs=pl.BlockSpec((BR, d_out), lambda i: (i, 0)),
        out_shape=jax.ShapeDtypeStruct((N_NODES, d_out), jnp.float32),
    )(q, cnt, b2)


def kernel(g, features, W1, b1, W2, b2):
    src = g[0].reshape(NW, NITER, CHUNK)
    dst = g[1].reshape(NW, NITER, CHUNK)
    nhid = W1.shape[1]
    nlabel = W2.shape[1]

    ones_c = jnp.ones((CHUNK,), jnp.float32)
    zeros_r = jnp.zeros((RPT,), jnp.float32)
    zeros_h = jnp.zeros((RPT, nhid), jnp.float32)
    zeros_o = jnp.zeros((RPT, nlabel), jnp.float32)

    counts = _sc_degrees(src, dst, ones_c, zeros_r)
    cnt = counts.reshape(4, NPAD).T

    h1 = _tc_linear1(features, cnt, W1)
    p = _sc_aggregate(h1, src, dst, zeros_h, nhid)
    h2 = _tc_mid(p, cnt, b1.reshape(1, nhid), W2)
    q = _sc_aggregate(h2, src, dst, zeros_o, nlabel, linear_tiling=True)
    return _tc_out(q, cnt, b2.reshape(1, nlabel), nlabel)


# speedup vs baseline: 1.2390x; 1.1002x over previous
import functools

import jax
import jax.numpy as jnp
from jax import lax
from jax.experimental import pallas as pl
from jax.experimental.pallas import tpu as pltpu
from jax.experimental.pallas import tpu_sc as plsc

N_NODES = 10000
N_EDGES = 320000
NC, NS = 2, 16
NW = NC * NS
EPW = N_EDGES // NW
CHUNK = 125
NITER = EPW // CHUNK
QUAD = 4
NQD = NITER // QUAD
PAIR = 2
NPAD = 10240
RPT = NPAD // NS
BR = 1000


def _sc_degrees(gr, ones_c, zeros_r):
    mesh = plsc.VectorSubcoreMesh(core_axis_name="c", subcore_axis_name="s")

    @functools.partial(
        pl.kernel,
        out_type=jax.ShapeDtypeStruct((NC, 2, NPAD), jnp.float32),
        mesh=mesh,
        scratch_types=[
            pltpu.VMEM((NITER, CHUNK), jnp.int32),
            pltpu.VMEM((NITER, CHUNK), jnp.int32),
            pltpu.VMEM((CHUNK,), jnp.float32),
            pltpu.VMEM_SHARED((NPAD,), jnp.float32),
            pltpu.VMEM_SHARED((NPAD,), jnp.float32),
        ] + [pltpu.SemaphoreType.DMA] * (2 * QUAD),
    )
    def deg_kernel(g_hbm, ones_hbm, zeros_hbm, out_hbm,
                   sidx, didx, ones_v, cs, cd, *sems):
        cid = lax.axis_index("c")
        sid = lax.axis_index("s")
        wid = sid * NC + cid
        pltpu.sync_copy(ones_hbm, ones_v)
        pltpu.sync_copy(g_hbm.at[0, wid], sidx)
        pltpu.sync_copy(g_hbm.at[1, wid], didx)
        pltpu.sync_copy(zeros_hbm, cs.at[pl.ds(sid * RPT, RPT)])
        pltpu.sync_copy(zeros_hbm, cd.at[pl.ds(sid * RPT, RPT)])
        plsc.subcore_barrier()

        def body(q, _):
            descs = []
            for k in range(QUAD):
                i = q * QUAD + k
                descs.append(pltpu.async_copy(
                    ones_v, cs.at[sidx.at[i]], sems[k], add=True))
                descs.append(pltpu.async_copy(
                    ones_v, cd.at[didx.at[i]], sems[QUAD + k], add=True))
            for de in descs:
                de.wait()
            return ()

        lax.fori_loop(0, NQD, body, ())
        plsc.subcore_barrier()
        pltpu.sync_copy(cs.at[pl.ds(sid * RPT, RPT)],
                        out_hbm.at[cid, 0, pl.ds(sid * RPT, RPT)])
        pltpu.sync_copy(cd.at[pl.ds(sid * RPT, RPT)],
                        out_hbm.at[cid, 1, pl.ds(sid * RPT, RPT)])

    return deg_kernel(gr, ones_c, zeros_r)


def _sc_aggregate(h, gr, zeros_rows, d, linear_tiling=False):
    mesh = plsc.VectorSubcoreMesh(core_axis_name="c", subcore_axis_name="s")
    params = (pltpu.CompilerParams(use_tc_tiling_on_sc=False)
              if linear_tiling else None)

    @functools.partial(
        pl.kernel,
        out_type=jax.ShapeDtypeStruct((NC, NPAD, d), jnp.float32),
        mesh=mesh,
        compiler_params=params,
        scratch_types=[
            pltpu.VMEM((PAIR, CHUNK), jnp.int32),
            pltpu.VMEM((PAIR, CHUNK), jnp.int32),
            pltpu.VMEM((PAIR, CHUNK, d), jnp.float32),
            pltpu.VMEM_SHARED((NPAD, d), jnp.float32),
            pltpu.SemaphoreType.DMA((PAIR,)),
            pltpu.SemaphoreType.DMA((PAIR,)),
            pltpu.SemaphoreType.DMA,
            pltpu.SemaphoreType.DMA,
        ],
    )
    def agg_kernel(h_hbm, g_hbm, zeros_hbm, out_hbm,
                   sidx, didx, rows, acc, gsem, ssem, isem_s, isem_d):
        cid = lax.axis_index("c")
        sid = lax.axis_index("s")
        wid = sid * NC + cid
        pltpu.sync_copy(zeros_hbm, acc.at[pl.ds(sid * RPT, RPT)])
        plsc.subcore_barrier()

        def gather(i, b):
            return pltpu.async_copy(
                h_hbm.at[sidx.at[b]], rows.at[b], gsem.at[b])

        def scatter(b):
            return pltpu.async_copy(
                rows.at[b], acc.at[didx.at[b]], ssem.at[b], add=True)

        def load_idx(i, b):
            pltpu.async_copy(g_hbm.at[0, wid, i], sidx.at[b], isem_s)
            pltpu.async_copy(g_hbm.at[1, wid, i], didx.at[b], isem_d)

        def wait_idx(b):
            pltpu.make_async_copy(g_hbm.at[0, wid, 0], sidx.at[b],
                                  isem_s).wait()
            pltpu.make_async_copy(g_hbm.at[1, wid, 0], didx.at[b],
                                  isem_d).wait()

        pltpu.sync_copy(g_hbm.at[0, wid, 0], sidx.at[0])
        pltpu.sync_copy(g_hbm.at[1, wid, 0], didx.at[0])
        gather(0, 0)

        def body(i, _):
            b = i % PAIR
            nb = (i + 1) % PAIR

            @pl.when(i >= PAIR - 1)
            def _():
                pltpu.make_async_copy(rows.at[nb], acc.at[didx.at[nb]],
                                      ssem.at[nb]).wait()

            @pl.when(i + 1 < NITER)
            def _():
                load_idx(i + 1, nb)

            pltpu.make_async_copy(h_hbm.at[sidx.at[b]], rows.at[b],
                                  gsem.at[b]).wait()
            scatter(b)

            @pl.when(i + 1 < NITER)
            def _():
                wait_idx(nb)
                gather(i + 1, nb)

            return ()

        lax.fori_loop(0, NITER, body, ())
        for j in range(NITER - PAIR + 1, NITER):
            pltpu.make_async_copy(rows.at[j % PAIR],
                                  acc.at[didx.at[j % PAIR]],
                                  ssem.at[j % PAIR]).wait()
        plsc.subcore_barrier()
        pltpu.sync_copy(acc.at[pl.ds(sid * RPT, RPT)],
                        out_hbm.at[cid, pl.ds(sid * RPT, RPT)])

    return agg_kernel(h, gr, zeros_rows)


def _norm_src(c):
    return lax.rsqrt(jnp.maximum(c[:, 0:1] + c[:, 2:3], 1.0))


def _norm_dst(c):
    return lax.rsqrt(jnp.maximum(c[:, 1:2] + c[:, 3:4], 1.0))


def _tc_linear1(x, cnt, w):
    d_in, d_out = w.shape

    def body(x_ref, c_ref, w_ref, o_ref):
        ns = _norm_src(c_ref[...])
        o_ref[...] = jnp.dot(x_ref[...] * ns, w_ref[...],
                             preferred_element_type=jnp.float32)

    return pl.pallas_call(
        body,
        grid=(N_NODES // BR,),
        in_specs=[
            pl.BlockSpec((BR, d_in), lambda i: (i, 0)),
            pl.BlockSpec((BR, 4), lambda i: (i, 0)),
            pl.BlockSpec((d_in, d_out), lambda i: (0, 0)),
        ],
        out_specs=pl.BlockSpec((BR, d_out), lambda i: (i, 0)),
        out_shape=jax.ShapeDtypeStruct((N_NODES, d_out), jnp.float32),
    )(x, cnt, w)


def _tc_mid(p, cnt, b1, w2):
    d_in, d_out = w2.shape

    def body(p_ref, c_ref, b_ref, w_ref, o_ref):
        c = c_ref[...]
        x = (p_ref[0] + p_ref[1]) * _norm_dst(c) + b_ref[...]
        x = jnp.maximum(x, 0.0)
        o_ref[...] = jnp.dot(x * _norm_src(c), w_ref[...],
                             preferred_element_type=jnp.float32)

    return pl.pallas_call(
        body,
        grid=(N_NODES // BR,),
        in_specs=[
            pl.BlockSpec((NC, BR, d_in), lambda i: (0, i, 0)),
            pl.BlockSpec((BR, 4), lambda i: (i, 0)),
            pl.BlockSpec((1, d_in), lambda i: (0, 0)),
            pl.BlockSpec((d_in, d_out), lambda i: (0, 0)),
        ],
        out_specs=pl.BlockSpec((BR, d_out), lambda i: (i, 0)),
        out_shape=jax.ShapeDtypeStruct((N_NODES, d_out), jnp.float32),
    )(p, cnt, b1, w2)


def _tc_out(q, cnt, b2, d_out):
    d = q.shape[-1]

    def body(q_ref, c_ref, b_ref, o_ref):
        t = (q_ref[0] + q_ref[1]) * _norm_dst(c_ref[...])
        o_ref[...] = t[:, :d_out] + b_ref[...]

    return pl.pallas_call(
        body,
        grid=(N_NODES // BR,),
        in_specs=[
            pl.BlockSpec((NC, BR, d), lambda i: (0, i, 0)),
            pl.BlockSpec((BR, 4), lambda i: (i, 0)),
            pl.BlockSpec((1, d_out), lambda i: (0, 0)),
        ],
        out_specs=pl.BlockSpec((BR, d_out), lambda i: (i, 0)),
        out_shape=jax.ShapeDtypeStruct((N_NODES, d_out), jnp.float32),
    )(q, cnt, b2)


def kernel(g, features, W1, b1, W2, b2):
    gr = g.reshape(2, NW, NITER, CHUNK)
    nhid = W1.shape[1]
    nlabel = W2.shape[1]

    ones_c = jnp.ones((CHUNK,), jnp.float32)
    zeros_r = jnp.zeros((RPT,), jnp.float32)
    zeros_h = jnp.zeros((RPT, nhid), jnp.float32)
    zeros_o = jnp.zeros((RPT, nlabel), jnp.float32)

    counts = _sc_degrees(gr, ones_c, zeros_r)
    cnt = counts.reshape(4, NPAD).T

    h1 = _tc_linear1(features, cnt, W1)
    p = _sc_aggregate(h1, gr, zeros_h, nhid)
    h2 = _tc_mid(p, cnt, b1.reshape(1, nhid), W2)
    q = _sc_aggregate(h2, gr, zeros_o, nlabel, linear_tiling=True)
    return _tc_out(q, cnt, b2.reshape(1, nlabel), nlabel)


# speedup vs baseline: 1.2695x; 1.0246x over previous
import functools

import jax
import jax.numpy as jnp
from jax import lax
from jax.experimental import pallas as pl
from jax.experimental.pallas import tpu as pltpu
from jax.experimental.pallas import tpu_sc as plsc

N_NODES = 10000
N_EDGES = 320000
NC, NS = 2, 16
NW = NC * NS
EPW = N_EDGES // NW
CHUNK = 125
NITER = EPW // CHUNK
QUAD = 4
NQD = NITER // QUAD
PAIR = 2
NPAD = 10240
RPT = NPAD // NS
BR = 2000


def _sc_degrees(gr, ones_c, zeros_r):
    mesh = plsc.VectorSubcoreMesh(core_axis_name="c", subcore_axis_name="s")

    @functools.partial(
        pl.kernel,
        out_type=jax.ShapeDtypeStruct((NC, 2, NPAD), jnp.float32),
        mesh=mesh,
        scratch_types=[
            pltpu.VMEM((NITER, CHUNK), jnp.int32),
            pltpu.VMEM((NITER, CHUNK), jnp.int32),
            pltpu.VMEM((CHUNK,), jnp.float32),
            pltpu.VMEM_SHARED((NPAD,), jnp.float32),
            pltpu.VMEM_SHARED((NPAD,), jnp.float32),
        ] + [pltpu.SemaphoreType.DMA] * (2 * QUAD),
    )
    def deg_kernel(g_hbm, ones_hbm, zeros_hbm, out_hbm,
                   sidx, didx, ones_v, cs, cd, *sems):
        cid = lax.axis_index("c")
        sid = lax.axis_index("s")
        wid = sid * NC + cid
        pltpu.sync_copy(ones_hbm, ones_v)
        pltpu.sync_copy(g_hbm.at[0, wid], sidx)
        pltpu.sync_copy(g_hbm.at[1, wid], didx)
        pltpu.sync_copy(zeros_hbm, cs.at[pl.ds(sid * RPT, RPT)])
        pltpu.sync_copy(zeros_hbm, cd.at[pl.ds(sid * RPT, RPT)])
        plsc.subcore_barrier()

        def body(q, _):
            descs = []
            for k in range(QUAD):
                i = q * QUAD + k
                descs.append(pltpu.async_copy(
                    ones_v, cs.at[sidx.at[i]], sems[k], add=True))
                descs.append(pltpu.async_copy(
                    ones_v, cd.at[didx.at[i]], sems[QUAD + k], add=True))
            for de in descs:
                de.wait()
            return ()

        lax.fori_loop(0, NQD, body, ())
        plsc.subcore_barrier()
        pltpu.sync_copy(cs.at[pl.ds(sid * RPT, RPT)],
                        out_hbm.at[cid, 0, pl.ds(sid * RPT, RPT)])
        pltpu.sync_copy(cd.at[pl.ds(sid * RPT, RPT)],
                        out_hbm.at[cid, 1, pl.ds(sid * RPT, RPT)])

    return deg_kernel(gr, ones_c, zeros_r)


def _sc_aggregate(h, gr, zeros_rows, d, linear_tiling=False):
    mesh = plsc.VectorSubcoreMesh(core_axis_name="c", subcore_axis_name="s")
    params = (pltpu.CompilerParams(use_tc_tiling_on_sc=False)
              if linear_tiling else None)

    @functools.partial(
        pl.kernel,
        out_type=jax.ShapeDtypeStruct((NC, NPAD, d), jnp.float32),
        mesh=mesh,
        compiler_params=params,
        scratch_types=[
            pltpu.VMEM((PAIR, CHUNK), jnp.int32),
            pltpu.VMEM((PAIR, CHUNK), jnp.int32),
            pltpu.VMEM((PAIR, CHUNK, d), jnp.float32),
            pltpu.VMEM_SHARED((NPAD, d), jnp.float32),
            pltpu.SemaphoreType.DMA((PAIR,)),
            pltpu.SemaphoreType.DMA((PAIR,)),
            pltpu.SemaphoreType.DMA,
            pltpu.SemaphoreType.DMA,
        ],
    )
    def agg_kernel(h_hbm, g_hbm, zeros_hbm, out_hbm,
                   sidx, didx, rows, acc, gsem, ssem, isem_s, isem_d):
        cid = lax.axis_index("c")
        sid = lax.axis_index("s")
        wid = sid * NC + cid
        pltpu.sync_copy(zeros_hbm, acc.at[pl.ds(sid * RPT, RPT)])
        plsc.subcore_barrier()

        def gather(i, b):
            return pltpu.async_copy(
                h_hbm.at[sidx.at[b]], rows.at[b], gsem.at[b])

        def scatter(b):
            return pltpu.async_copy(
                rows.at[b], acc.at[didx.at[b]], ssem.at[b], add=True)

        def load_idx(i, b):
            pltpu.async_copy(g_hbm.at[0, wid, i], sidx.at[b], isem_s)
            pltpu.async_copy(g_hbm.at[1, wid, i], didx.at[b], isem_d)

        def wait_idx(b):
            pltpu.make_async_copy(g_hbm.at[0, wid, 0], sidx.at[b],
                                  isem_s).wait()
            pltpu.make_async_copy(g_hbm.at[1, wid, 0], didx.at[b],
                                  isem_d).wait()

        pltpu.sync_copy(g_hbm.at[0, wid, 0], sidx.at[0])
        pltpu.sync_copy(g_hbm.at[1, wid, 0], didx.at[0])
        gather(0, 0)

        def body(i, _):
            b = i % PAIR
            nb = (i + 1) % PAIR

            @pl.when(i >= PAIR - 1)
            def _():
                pltpu.make_async_copy(rows.at[nb], acc.at[didx.at[nb]],
                                      ssem.at[nb]).wait()

            @pl.when(i + 1 < NITER)
            def _():
                load_idx(i + 1, nb)

            pltpu.make_async_copy(h_hbm.at[sidx.at[b]], rows.at[b],
                                  gsem.at[b]).wait()
            scatter(b)

            @pl.when(i + 1 < NITER)
            def _():
                wait_idx(nb)
                gather(i + 1, nb)

            return ()

        lax.fori_loop(0, NITER, body, ())
        for j in range(NITER - PAIR + 1, NITER):
            pltpu.make_async_copy(rows.at[j % PAIR],
                                  acc.at[didx.at[j % PAIR]],
                                  ssem.at[j % PAIR]).wait()
        plsc.subcore_barrier()
        pltpu.sync_copy(acc.at[pl.ds(sid * RPT, RPT)],
                        out_hbm.at[cid, pl.ds(sid * RPT, RPT)])

    return agg_kernel(h, gr, zeros_rows)


def _norm_src(c):
    return lax.rsqrt(jnp.maximum(c[:, 0:1] + c[:, 2:3], 1.0))


def _norm_dst(c):
    return lax.rsqrt(jnp.maximum(c[:, 1:2] + c[:, 3:4], 1.0))


def _tc_linear1(x, cnt, w):
    d_in, d_out = w.shape

    def body(x_ref, c_ref, w_ref, o_ref):
        ns = _norm_src(c_ref[...])
        o_ref[...] = jnp.dot(x_ref[...] * ns, w_ref[...],
                             preferred_element_type=jnp.float32)

    return pl.pallas_call(
        body,
        grid=(N_NODES // BR,),
        in_specs=[
            pl.BlockSpec((BR, d_in), lambda i: (i, 0)),
            pl.BlockSpec((BR, 4), lambda i: (i, 0)),
            pl.BlockSpec((d_in, d_out), lambda i: (0, 0)),
        ],
        out_specs=pl.BlockSpec((BR, d_out), lambda i: (i, 0)),
        out_shape=jax.ShapeDtypeStruct((N_NODES, d_out), jnp.float32),
    )(x, cnt, w)


def _tc_mid(p, cnt, b1, w2):
    d_in, d_out = w2.shape

    def body(p_ref, c_ref, b_ref, w_ref, o_ref):
        c = c_ref[...]
        x = (p_ref[0] + p_ref[1]) * _norm_dst(c) + b_ref[...]
        x = jnp.maximum(x, 0.0)
        o_ref[...] = jnp.dot(x * _norm_src(c), w_ref[...],
                             preferred_element_type=jnp.float32)

    return pl.pallas_call(
        body,
        grid=(N_NODES // BR,),
        in_specs=[
            pl.BlockSpec((NC, BR, d_in), lambda i: (0, i, 0)),
            pl.BlockSpec((BR, 4), lambda i: (i, 0)),
            pl.BlockSpec((1, d_in), lambda i: (0, 0)),
            pl.BlockSpec((d_in, d_out), lambda i: (0, 0)),
        ],
        out_specs=pl.BlockSpec((BR, d_out), lambda i: (i, 0)),
        out_shape=jax.ShapeDtypeStruct((N_NODES, d_out), jnp.float32),
    )(p, cnt, b1, w2)


def _tc_out(q, cnt, b2, d_out):
    d = q.shape[-1]

    def body(q_ref, c_ref, b_ref, o_ref):
        t = (q_ref[0] + q_ref[1]) * _norm_dst(c_ref[...])
        o_ref[...] = t[:, :d_out] + b_ref[...]

    return pl.pallas_call(
        body,
        grid=(N_NODES // BR,),
        in_specs=[
            pl.BlockSpec((NC, BR, d), lambda i: (0, i, 0)),
            pl.BlockSpec((BR, 4), lambda i: (i, 0)),
            pl.BlockSpec((1, d_out), lambda i: (0, 0)),
        ],
        out_specs=pl.BlockSpec((BR, d_out), lambda i: (i, 0)),
        out_shape=jax.ShapeDtypeStruct((N_NODES, d_out), jnp.float32),
    )(q, cnt, b2)


def kernel(g, features, W1, b1, W2, b2):
    gr = g.reshape(2, NW, NITER, CHUNK)
    nhid = W1.shape[1]
    nlabel = W2.shape[1]

    ones_c = jnp.ones((CHUNK,), jnp.float32)
    zeros_r = jnp.zeros((RPT,), jnp.float32)
    zeros_h = jnp.zeros((RPT, nhid), jnp.float32)
    zeros_o = jnp.zeros((RPT, nlabel), jnp.float32)

    counts = _sc_degrees(gr, ones_c, zeros_r)
    cnt = counts.reshape(4, NPAD).T

    h1 = _tc_linear1(features, cnt, W1)
    p = _sc_aggregate(h1, gr, zeros_h, nhid)
    h2 = _tc_mid(p, cnt, b1.reshape(1, nhid), W2)
    q = _sc_aggregate(h2, gr, zeros_o, nlabel, linear_tiling=True)
    return _tc_out(q, cnt, b2.reshape(1, nlabel), nlabel)


# speedup vs baseline: 1.2719x; 1.0019x over previous
import functools

import jax
import jax.numpy as jnp
from jax import lax
from jax.experimental import pallas as pl
from jax.experimental.pallas import tpu as pltpu
from jax.experimental.pallas import tpu_sc as plsc

N_NODES = 10000
N_EDGES = 320000
NC, NS = 2, 16
NW = NC * NS
EPW = N_EDGES // NW
CHUNK = 125
NITER = EPW // CHUNK
QUAD = 4
NQD = NITER // QUAD
PAIR = 2
NPAD = 10240
RPT = NPAD // NS
BR = 2000


def _sc_degrees(gr, ones_c, zeros_r):
    mesh = plsc.VectorSubcoreMesh(core_axis_name="c", subcore_axis_name="s")

    @functools.partial(
        pl.kernel,
        out_type=jax.ShapeDtypeStruct((NC, 2, NPAD), jnp.float32),
        mesh=mesh,
        compiler_params=pltpu.CompilerParams(use_tc_tiling_on_sc=False),
        scratch_types=[
            pltpu.VMEM((NITER, CHUNK), jnp.int32),
            pltpu.VMEM((NITER, CHUNK), jnp.int32),
            pltpu.VMEM((CHUNK,), jnp.float32),
            pltpu.VMEM_SHARED((NPAD,), jnp.float32),
            pltpu.VMEM_SHARED((NPAD,), jnp.float32),
        ] + [pltpu.SemaphoreType.DMA] * (2 * QUAD),
    )
    def deg_kernel(g_hbm, ones_hbm, zeros_hbm, out_hbm,
                   sidx, didx, ones_v, cs, cd, *sems):
        cid = lax.axis_index("c")
        sid = lax.axis_index("s")
        wid = sid * NC + cid
        pltpu.sync_copy(ones_hbm, ones_v)
        pltpu.sync_copy(g_hbm.at[0, wid], sidx)
        pltpu.sync_copy(g_hbm.at[1, wid], didx)
        pltpu.sync_copy(zeros_hbm, cs.at[pl.ds(sid * RPT, RPT)])
        pltpu.sync_copy(zeros_hbm, cd.at[pl.ds(sid * RPT, RPT)])
        plsc.subcore_barrier()

        def body(q, _):
            descs = []
            for k in range(QUAD):
                i = q * QUAD + k
                descs.append(pltpu.async_copy(
                    ones_v, cs.at[sidx.at[i]], sems[k], add=True))
                descs.append(pltpu.async_copy(
                    ones_v, cd.at[didx.at[i]], sems[QUAD + k], add=True))
            for de in descs:
                de.wait()
            return ()

        lax.fori_loop(0, NQD, body, ())
        plsc.subcore_barrier()
        pltpu.sync_copy(cs.at[pl.ds(sid * RPT, RPT)],
                        out_hbm.at[cid, 0, pl.ds(sid * RPT, RPT)])
        pltpu.sync_copy(cd.at[pl.ds(sid * RPT, RPT)],
                        out_hbm.at[cid, 1, pl.ds(sid * RPT, RPT)])

    return deg_kernel(gr, ones_c, zeros_r)


def _sc_aggregate(h, gr, zeros_rows, d, linear_tiling=False):
    mesh = plsc.VectorSubcoreMesh(core_axis_name="c", subcore_axis_name="s")
    params = (pltpu.CompilerParams(use_tc_tiling_on_sc=False)
              if linear_tiling else None)

    @functools.partial(
        pl.kernel,
        out_type=jax.ShapeDtypeStruct((NC, NPAD, d), jnp.float32),
        mesh=mesh,
        compiler_params=params,
        scratch_types=[
            pltpu.VMEM((PAIR, CHUNK), jnp.int32),
            pltpu.VMEM((PAIR, CHUNK), jnp.int32),
            pltpu.VMEM((PAIR, CHUNK, d), jnp.float32),
            pltpu.VMEM_SHARED((NPAD, d), jnp.float32),
            pltpu.SemaphoreType.DMA((PAIR,)),
            pltpu.SemaphoreType.DMA((PAIR,)),
            pltpu.SemaphoreType.DMA,
            pltpu.SemaphoreType.DMA,
        ],
    )
    def agg_kernel(h_hbm, g_hbm, zeros_hbm, out_hbm,
                   sidx, didx, rows, acc, gsem, ssem, isem_s, isem_d):
        cid = lax.axis_index("c")
        sid = lax.axis_index("s")
        wid = sid * NC + cid
        pltpu.sync_copy(zeros_hbm, acc.at[pl.ds(sid * RPT, RPT)])
        plsc.subcore_barrier()

        def gather(i, b):
            return pltpu.async_copy(
                h_hbm.at[sidx.at[b]], rows.at[b], gsem.at[b])

        def scatter(b):
            return pltpu.async_copy(
                rows.at[b], acc.at[didx.at[b]], ssem.at[b], add=True)

        def load_idx(i, b):
            pltpu.async_copy(g_hbm.at[0, wid, i], sidx.at[b], isem_s)
            pltpu.async_copy(g_hbm.at[1, wid, i], didx.at[b], isem_d)

        def wait_idx(b):
            pltpu.make_async_copy(g_hbm.at[0, wid, 0], sidx.at[b],
                                  isem_s).wait()
            pltpu.make_async_copy(g_hbm.at[1, wid, 0], didx.at[b],
                                  isem_d).wait()

        pltpu.sync_copy(g_hbm.at[0, wid, 0], sidx.at[0])
        pltpu.sync_copy(g_hbm.at[1, wid, 0], didx.at[0])
        gather(0, 0)

        def body(i, _):
            b = i % PAIR
            nb = (i + 1) % PAIR

            @pl.when(i >= PAIR - 1)
            def _():
                pltpu.make_async_copy(rows.at[nb], acc.at[didx.at[nb]],
                                      ssem.at[nb]).wait()

            @pl.when(i + 1 < NITER)
            def _():
                load_idx(i + 1, nb)

            pltpu.make_async_copy(h_hbm.at[sidx.at[b]], rows.at[b],
                                  gsem.at[b]).wait()
            scatter(b)

            @pl.when(i + 1 < NITER)
            def _():
                wait_idx(nb)
                gather(i + 1, nb)

            return ()

        lax.fori_loop(0, NITER, body, ())
        for j in range(NITER - PAIR + 1, NITER):
            pltpu.make_async_copy(rows.at[j % PAIR],
                                  acc.at[didx.at[j % PAIR]],
                                  ssem.at[j % PAIR]).wait()
        plsc.subcore_barrier()
        pltpu.sync_copy(acc.at[pl.ds(sid * RPT, RPT)],
                        out_hbm.at[cid, pl.ds(sid * RPT, RPT)])

    return agg_kernel(h, gr, zeros_rows)


def _norm_src(c):
    return lax.rsqrt(jnp.maximum(c[:, 0:1] + c[:, 2:3], 1.0))


def _norm_dst(c):
    return lax.rsqrt(jnp.maximum(c[:, 1:2] + c[:, 3:4], 1.0))


def _tc_linear1(x, cnt, w):
    d_in, d_out = w.shape

    def body(x_ref, c_ref, w_ref, o_ref):
        ns = _norm_src(c_ref[...])
        o_ref[...] = jnp.dot(x_ref[...] * ns, w_ref[...],
                             preferred_element_type=jnp.float32)

    return pl.pallas_call(
        body,
        grid=(N_NODES // BR,),
        in_specs=[
            pl.BlockSpec((BR, d_in), lambda i: (i, 0)),
            pl.BlockSpec((BR, 4), lambda i: (i, 0)),
            pl.BlockSpec((d_in, d_out), lambda i: (0, 0)),
        ],
        out_specs=pl.BlockSpec((BR, d_out), lambda i: (i, 0)),
        out_shape=jax.ShapeDtypeStruct((N_NODES, d_out), jnp.float32),
    )(x, cnt, w)


def _tc_mid(p, cnt, b1, w2):
    d_in, d_out = w2.shape

    def body(p_ref, c_ref, b_ref, w_ref, o_ref):
        c = c_ref[...]
        x = (p_ref[0] + p_ref[1]) * _norm_dst(c) + b_ref[...]
        x = jnp.maximum(x, 0.0)
        o_ref[...] = jnp.dot(x * _norm_src(c), w_ref[...],
                             preferred_element_type=jnp.float32)

    return pl.pallas_call(
        body,
        grid=(N_NODES // BR,),
        in_specs=[
            pl.BlockSpec((NC, BR, d_in), lambda i: (0, i, 0)),
            pl.BlockSpec((BR, 4), lambda i: (i, 0)),
            pl.BlockSpec((1, d_in), lambda i: (0, 0)),
            pl.BlockSpec((d_in, d_out), lambda i: (0, 0)),
        ],
        out_specs=pl.BlockSpec((BR, d_out), lambda i: (i, 0)),
        out_shape=jax.ShapeDtypeStruct((N_NODES, d_out), jnp.float32),
    )(p, cnt, b1, w2)


def _tc_out(q, cnt, b2, d_out):
    d = q.shape[-1]

    def body(q_ref, c_ref, b_ref, o_ref):
        t = (q_ref[0] + q_ref[1]) * _norm_dst(c_ref[...])
        o_ref[...] = t[:, :d_out] + b_ref[...]

    return pl.pallas_call(
        body,
        grid=(N_NODES // BR,),
        in_specs=[
            pl.BlockSpec((NC, BR, d), lambda i: (0, i, 0)),
            pl.BlockSpec((BR, 4), lambda i: (i, 0)),
            pl.BlockSpec((1, d_out), lambda i: (0, 0)),
        ],
        out_specs=pl.BlockSpec((BR, d_out), lambda i: (i, 0)),
        out_shape=jax.ShapeDtypeStruct((N_NODES, d_out), jnp.float32),
    )(q, cnt, b2)


def kernel(g, features, W1, b1, W2, b2):
    gr = g.reshape(2, NW, NITER, CHUNK)
    nhid = W1.shape[1]
    nlabel = W2.shape[1]

    ones_c = jnp.ones((CHUNK,), jnp.float32)
    zeros_r = jnp.zeros((RPT,), jnp.float32)
    zeros_h = jnp.zeros((RPT, nhid), jnp.float32)
    zeros_o = jnp.zeros((RPT, nlabel), jnp.float32)

    counts = _sc_degrees(gr, ones_c, zeros_r)
    cnt = counts.reshape(4, NPAD).T

    h1 = _tc_linear1(features, cnt, W1)
    p = _sc_aggregate(h1, gr, zeros_h, nhid, linear_tiling=True)
    h2 = _tc_mid(p, cnt, b1.reshape(1, nhid), W2)
    q = _sc_aggregate(h2, gr, zeros_o, nlabel, linear_tiling=True)
    return _tc_out(q, cnt, b2.reshape(1, nlabel), nlabel)
